# baseline (speedup 1.0000x reference)
import functools

import jax
import jax.numpy as jnp
from jax import lax
from jax.experimental import pallas as pl
from jax.experimental.pallas import tpu as pltpu
from jax.experimental.pallas import tpu_sc as plsc

EMBED = 32
LANES = 16
NC, NS = 2, 16
NW = NC * NS
BBLK = 128


def _sc_embed_t(idx_t, table2, pos):
    seq, batch = idx_t.shape

    mesh = plsc.VectorSubcoreMesh(core_axis_name="c", subcore_axis_name="s")

    @functools.partial(
        pl.kernel,
        out_type=jax.ShapeDtypeStruct((seq, batch, EMBED), jnp.float32),
        mesh=mesh,
        scratch_types=[
            pltpu.VMEM((seq, EMBED), jnp.float32),
            pltpu.VMEM((BBLK,), jnp.int32),
            pltpu.VMEM((BBLK,), jnp.int32),
            pltpu.VMEM((BBLK, 128), jnp.float32),
            pltpu.VMEM((BBLK, EMBED), jnp.float32),
            pltpu.SemaphoreType.DMA,
        ],
    )
    def k(idx_hbm, tok_hbm, pos_hbm, out_hbm, pos_v, t_v, r_v, g_v, o_v, sem):
        wid = lax.axis_index("s") * NC + lax.axis_index("c")
        b0 = wid * BBLK
        pltpu.sync_copy(pos_hbm, pos_v)

        def step(s, carry):
            pltpu.sync_copy(idx_hbm.at[s, pl.ds(b0, BBLK)], t_v)
            tvs = []
            for l in range(BBLK // LANES):
                tv = t_v[pl.ds(l * LANES, LANES)]
                tvs.append(tv)
                r_v[pl.ds(l * LANES, LANES)] = lax.shift_right_logical(tv, 2)
            pltpu.async_copy(tok_hbm.at[r_v], g_v, sem).wait()

            pv0 = pos_v[s, pl.ds(0, LANES)]
            pv1 = pos_v[s, pl.ds(LANES, LANES)]
            for j in range(BBLK):
                q = tvs[j // LANES][j % LANES] & 3
                c = lax.mul(q, 32)
                o_v[j, pl.ds(0, LANES)] = g_v[j, pl.ds(c, LANES)] + pv0
                o_v[j, pl.ds(LANES, LANES)] = (
                    g_v[j, pl.ds(c + LANES, LANES)] + pv1
                )
            pltpu.sync_copy(o_v, out_hbm.at[s, pl.ds(b0, BBLK)])
            return carry

        lax.fori_loop(0, seq, step, 0)

    return k(idx_t, table2, pos)


def kernel(inputs, token_table, pos_table):
    b, s = inputs.shape
    idx_t = inputs.astype(jnp.int32).T
    table2 = jnp.reshape(token_table, (-1, 128))
    out_t = _sc_embed_t(idx_t, table2, pos_table[:s])
    return out_t.transpose(1, 0, 2)
